# baseline (speedup 1.0000x reference)
import functools
import jax
import jax.numpy as jnp
from jax import lax
from jax.experimental import pallas as pl
from jax.experimental.pallas import tpu as pltpu
from jax.experimental.pallas import tpu_sc as plsc

_SINK = 5
_TOPK_MAIN = 3
_CHUNK = 128
_TILE = 256
_NEG_INF = float("-inf")


def _tc_body(q_ref, k_ref, wq_ref, wk_ref, g_ref, bln_ref, wu_ref, wv_ref,
             bu_ref, bv_ref, wo_ref, bo_ref, logits_ref, x_ref, wau_ref,
             wav_ref):
    H = wq_ref.shape[1]
    Jc = k_ref.shape[1]
    tile = q_ref.shape[1]
    f32 = jnp.float32

    Qt = q_ref[0]
    Kb = k_ref[0]
    qp = jnp.dot(Qt, wq_ref[...], preferred_element_type=f32)
    kp = jnp.dot(Kb, wk_ref[...], preferred_element_type=f32)

    g = g_ref[...]
    bln = bln_ref[...]
    Wu = wu_ref[...]
    Wv = wv_ref[...]
    g_q = g[:, :H]
    g_k = g[:, H:2 * H]
    g_m = g[:, 2 * H:]

    su = jnp.dot(g, Wu, preferred_element_type=f32)
    sv = jnp.dot(g, Wv, preferred_element_type=f32)
    cu = jnp.dot(bln, Wu, preferred_element_type=f32) + bu_ref[...]
    cv = jnp.dot(bln, Wv, preferred_element_type=f32) + bv_ref[...]

    ones_h = jnp.ones((1, H), dtype=f32)
    qp2 = qp * qp
    kp2 = kp * kp
    s1q = jnp.sum(qp, axis=1, keepdims=True)
    s2q = jnp.sum(qp2, axis=1, keepdims=True)
    dn = (((1,), (1,)), ((), ()))
    s1k = lax.dot_general(ones_h, kp, dn, preferred_element_type=f32)
    s2k = lax.dot_general(ones_h, kp2, dn, preferred_element_type=f32)
    G1 = lax.dot_general(qp, kp, dn, preferred_element_type=f32)
    G2 = lax.dot_general(qp2, kp2, dn, preferred_element_type=f32)
    inv_d = 1.0 / (3.0 * H)
    mu = (s1q + s1k + G1) * inv_d
    ex2 = (s2q + s2k + G2) * inv_d
    rstd = lax.rsqrt(ex2 - mu * mu + 1e-5)

    qs = qp * g_q
    ks = kp * g_k
    kg = kp * g_m
    Wu_q, Wu_k, Wu_m = Wu[:H], Wu[H:2 * H], Wu[2 * H:]
    Wv_q, Wv_k, Wv_m = Wv[:H], Wv[H:2 * H], Wv[2 * H:]
    Aq_u = jnp.dot(qs, Wu_q, preferred_element_type=f32)
    Aq_v = jnp.dot(qs, Wv_q, preferred_element_type=f32)
    Ak_u = jnp.dot(ks, Wu_k, preferred_element_type=f32)
    Ak_v = jnp.dot(ks, Wv_k, preferred_element_type=f32)

    XW = x_ref.shape[1]
    pad = XW - H - 8
    x_ref[:, H:] = (lax.broadcasted_iota(jnp.int32, (tile, pad + 8), 1)
                    == 1).astype(f32)
    wau_ref[:H] = Wu_m
    wav_ref[:H] = Wv_m
    wau_ref[H:H + 1, :] = su
    wav_ref[H:H + 1, :] = sv
    zfill = jnp.zeros((pad + 6, H), dtype=f32)
    wau_ref[H + 2:] = zfill
    wav_ref[H + 2:] = zfill

    wo = wo_ref[...]
    bos = bo_ref[0, 0]
    cols = []
    for j in range(Jc):
        x_ref[:, :H] = qp * kg[j:j + 1, :]
        x_ref[:, H:H + 1] = -mu[:, j:j + 1]
        wau_ref[H + 1:H + 2, :] = Ak_u[j:j + 1, :]
        wav_ref[H + 1:H + 2, :] = Ak_v[j:j + 1, :]
        Xv = x_ref[...]
        Mu2 = jnp.dot(Xv, wau_ref[...], preferred_element_type=f32)
        Mv2 = jnp.dot(Xv, wav_ref[...], preferred_element_type=f32)
        rj = rstd[:, j:j + 1]
        U = rj * (Aq_u + Mu2) + cu
        V = rj * (Aq_v + Mv2) + cv
        geluV = 0.5 * V * (1.0 + lax.erf(V * 0.7071067811865476))
        Z = U * geluV
        cols.append(lax.dot_general(Z, wo, dn, preferred_element_type=f32))
    logits_ref[0] = jnp.concatenate(cols, axis=1) + bos


def _tc_logits(Q, K, Wq, Wk, ln_g, ln_b, Wu, bu, Wv, bv, Wo, bo):
    B, T, QD = Q.shape
    Jc = K.shape[1]
    KD = K.shape[2]
    H = Wq.shape[1]
    tile = _TILE
    grid = (B, T // tile)

    g2 = ln_g.reshape(1, -1)
    b2 = ln_b.reshape(1, -1)
    bu2 = bu.reshape(1, -1)
    bv2 = bv.reshape(1, -1)
    wo2 = Wo.reshape(1, -1)
    bo2 = bo.reshape(1, 1)

    full = lambda shape: pl.BlockSpec(shape, lambda b, t: (0,) * len(shape))
    return pl.pallas_call(
        _tc_body,
        grid=grid,
        in_specs=[
            pl.BlockSpec((1, tile, QD), lambda b, t: (b, t, 0)),
            pl.BlockSpec((1, Jc, KD), lambda b, t: (b, 0, 0)),
            full((QD, H)),
            full((KD, H)),
            full((1, 3 * H)),
            full((1, 3 * H)),
            full((3 * H, H)),
            full((3 * H, H)),
            full((1, H)),
            full((1, H)),
            full((1, H)),
            full((1, 1)),
        ],
        out_specs=pl.BlockSpec((1, tile, Jc), lambda b, t: (b, t, 0)),
        out_shape=jax.ShapeDtypeStruct((B, T, Jc), jnp.float32),
        scratch_shapes=[
            pltpu.VMEM((tile, 384), jnp.float32),
            pltpu.VMEM((384, H), jnp.float32),
            pltpu.VMEM((384, H), jnp.float32),
        ],
    )(Q, K, Wq, Wk, g2, b2, Wu, Wv, bu2, bv2, wo2, bo2)


def _sc_bias_call(ROWS, T, Jc, Lctx):
    NC, NS, L = 2, 16, 16
    NW = NC * NS
    rows_per = ROWS // NW
    RB = 32
    nbatch = rows_per // RB
    mesh = plsc.VectorSubcoreMesh(core_axis_name="c", subcore_axis_name="s")

    @functools.partial(
        pl.kernel, mesh=mesh,
        out_type=jax.ShapeDtypeStruct((ROWS, Lctx), jnp.float32),
        compiler_params=pltpu.CompilerParams(needs_layout_passes=False),
        scratch_types=[
            pltpu.VMEM((rows_per, Jc), jnp.float32),
            pltpu.VMEM((RB, Lctx), jnp.float32),
            pltpu.VMEM((L,), jnp.float32),
        ],
    )
    def k(logits_hbm, bias_hbm, lbuf, obuf, fscr):
        wid = lax.axis_index("s") * NC + lax.axis_index("c")
        base = wid * rows_per
        pltpu.sync_copy(logits_hbm.at[pl.ds(base, rows_per)], lbuf)

        lane = lax.broadcasted_iota(jnp.int32, (L,), 0)
        negv = jnp.full((L,), _NEG_INF, jnp.float32)
        zerov = jnp.zeros((L,), jnp.float32)
        i15 = jnp.full((L,), L - 1, jnp.int32)

        def splat_f(vec, idxv):
            fscr[...] = vec
            return plsc.load_gather(fscr, [idxv])

        def splat_last_f(vec):
            return splat_f(vec, i15)

        def batch_body(bi, carry):
            def row_body(rr, c2):
                r = bi * RB + rr
                lvec = lbuf[r]
                t = lax.rem(base + r, T)
                mxv = splat_last_f(plsc.cummax(lvec))
                e = jnp.exp(lvec - mxv)
                sv = splat_last_f(plsc.cumsum(e))
                w = e / sv
                pcb = -8.0 * (1.0 - w)
                tail_ok = jnp.logical_and(lane >= _SINK, lane - _SINK <= t)
                masked = jnp.where(tail_ok, lvec, negv)
                skeys, _svals = plsc.sort_key_val(masked, lane,
                                                  descending=True)
                thr = splat_f(skeys, jnp.full((L,), _TOPK_MAIN - 1,
                                              jnp.int32))
                sel = jnp.logical_or(lane < _SINK, masked >= thr)
                vals = jnp.where(sel, zerov, pcb)
                cvals = jnp.where(lane * _CHUNK <= t, vals, negv)
                fscr[...] = cvals
                for c in range(Jc):
                    idx = jnp.full((L,), c, jnp.int32)
                    splat = plsc.load_gather(fscr, [idx])
                    for g in range(_CHUNK // L):
                        obuf[rr, pl.ds(c * _CHUNK + g * L, L)] = splat
                cb = lax.shift_right_logical(t, 7)
                splat_b = plsc.load_gather(fscr, [jnp.full((L,), cb, jnp.int32)])
                cb128 = cb * _CHUNK
                for g in range(_CHUNK // L):
                    colv = lane + (cb128 + g * L)
                    obuf[rr, pl.ds(cb128 + g * L, L)] = jnp.where(
                        colv <= t, splat_b, negv)
                return c2

            lax.fori_loop(0, RB, row_body, 0, unroll=False)
            pltpu.sync_copy(obuf, bias_hbm.at[pl.ds(base + bi * RB, RB)])
            return carry

        lax.fori_loop(0, nbatch, batch_body, 0, unroll=False)

    return k


def kernel(Q, K, chunk_len, L_ctx, sink_blocks, Wq, Wk, ln_g, ln_b,
           Wu, bu, Wv, bv, Wo, bo):
    B, T, _ = Q.shape
    Jc = K.shape[1]
    Lctx = Jc * _CHUNK
    logits = _tc_logits(Q, K, Wq, Wk, ln_g, ln_b, Wu, bu, Wv, bv, Wo, bo)
    sc = _sc_bias_call(B * T, T, Jc, Lctx)
    bias = sc(logits.reshape(B * T, Jc)).reshape(B, T, Lctx)
    return logits, bias


# speedup vs baseline: 1.0728x; 1.0728x over previous
import functools
import jax
import jax.numpy as jnp
from jax import lax
from jax.experimental import pallas as pl
from jax.experimental.pallas import tpu as pltpu
from jax.experimental.pallas import tpu_sc as plsc

_SINK = 5
_TOPK_MAIN = 3
_CHUNK = 128
_TILE = 256
_NEG_INF = float("-inf")


def _tc_body(q_ref, k_ref, wq_ref, wk_ref, g_ref, bln_ref, wu_ref, wv_ref,
             bu_ref, bv_ref, wo_ref, bo_ref, logits_ref):
    H = wq_ref.shape[1]
    Jc = k_ref.shape[1]
    tile = q_ref.shape[1]
    f32 = jnp.float32

    Qt = q_ref[0]
    Kb = k_ref[0]
    qp = jnp.dot(Qt, wq_ref[...], preferred_element_type=f32)
    kp = jnp.dot(Kb, wk_ref[...], preferred_element_type=f32)

    g = g_ref[...]
    bln = bln_ref[...]
    Wu = wu_ref[...]
    Wv = wv_ref[...]
    g_q = g[:, :H]
    g_k = g[:, H:2 * H]
    g_m = g[:, 2 * H:]

    su = jnp.dot(g, Wu, preferred_element_type=f32)
    sv = jnp.dot(g, Wv, preferred_element_type=f32)
    cu = jnp.dot(bln, Wu, preferred_element_type=f32) + bu_ref[...]
    cv = jnp.dot(bln, Wv, preferred_element_type=f32) + bv_ref[...]

    ones_h = jnp.ones((1, H), dtype=f32)
    qp2 = qp * qp
    kp2 = kp * kp
    s1q = jnp.sum(qp, axis=1, keepdims=True)
    s2q = jnp.sum(qp2, axis=1, keepdims=True)
    dn = (((1,), (1,)), ((), ()))
    s1k = lax.dot_general(ones_h, kp, dn, preferred_element_type=f32)
    s2k = lax.dot_general(ones_h, kp2, dn, preferred_element_type=f32)
    G1 = lax.dot_general(qp, kp, dn, preferred_element_type=f32)
    G2 = lax.dot_general(qp2, kp2, dn, preferred_element_type=f32)
    inv_d = 1.0 / (3.0 * H)
    mu = (s1q + s1k + G1) * inv_d
    ex2 = (s2q + s2k + G2) * inv_d
    rstd = lax.rsqrt(ex2 - mu * mu + 1e-5)

    qs = qp * g_q
    ks = kp * g_k
    kg = kp * g_m
    Wu_q, Wu_k, Wu_m = Wu[:H], Wu[H:2 * H], Wu[2 * H:]
    Wv_q, Wv_k, Wv_m = Wv[:H], Wv[H:2 * H], Wv[2 * H:]
    Aq_u = jnp.dot(qs, Wu_q, preferred_element_type=f32)
    Aq_v = jnp.dot(qs, Wv_q, preferred_element_type=f32)
    Ak_u = jnp.dot(ks, Wu_k, preferred_element_type=f32)
    Ak_v = jnp.dot(ks, Wv_k, preferred_element_type=f32)

    wo = wo_ref[...]
    bos = bo_ref[0, 0]
    cols = []
    for j in range(Jc):
        xm = qp * kg[j:j + 1, :]
        Pu = jnp.dot(xm, Wu_m, preferred_element_type=f32)
        Pv = jnp.dot(xm, Wv_m, preferred_element_type=f32)
        rj = rstd[:, j:j + 1]
        muj = mu[:, j:j + 1]
        U = rj * (Aq_u + Ak_u[j:j + 1, :] + Pu - muj * su) + cu
        V = rj * (Aq_v + Ak_v[j:j + 1, :] + Pv - muj * sv) + cv
        geluV = 0.5 * V * (1.0 + lax.erf(V * 0.7071067811865476))
        Z = U * geluV
        cols.append(lax.dot_general(Z, wo, dn, preferred_element_type=f32))
    logits_ref[0] = jnp.concatenate(cols, axis=1) + bos


def _tc_logits(Q, K, Wq, Wk, ln_g, ln_b, Wu, bu, Wv, bv, Wo, bo):
    B, T, QD = Q.shape
    Jc = K.shape[1]
    KD = K.shape[2]
    H = Wq.shape[1]
    tile = _TILE
    grid = (B, T // tile)

    g2 = ln_g.reshape(1, -1)
    b2 = ln_b.reshape(1, -1)
    bu2 = bu.reshape(1, -1)
    bv2 = bv.reshape(1, -1)
    wo2 = Wo.reshape(1, -1)
    bo2 = bo.reshape(1, 1)

    full = lambda shape: pl.BlockSpec(shape, lambda b, t: (0,) * len(shape))
    return pl.pallas_call(
        _tc_body,
        grid=grid,
        in_specs=[
            pl.BlockSpec((1, tile, QD), lambda b, t: (b, t, 0)),
            pl.BlockSpec((1, Jc, KD), lambda b, t: (b, 0, 0)),
            full((QD, H)),
            full((KD, H)),
            full((1, 3 * H)),
            full((1, 3 * H)),
            full((3 * H, H)),
            full((3 * H, H)),
            full((1, H)),
            full((1, H)),
            full((1, H)),
            full((1, 1)),
        ],
        out_specs=pl.BlockSpec((1, tile, Jc), lambda b, t: (b, t, 0)),
        out_shape=jax.ShapeDtypeStruct((B, T, Jc), jnp.float32),
    )(Q, K, Wq, Wk, g2, b2, Wu, Wv, bu2, bv2, wo2, bo2)


def _sc_bias_call(ROWS, T, Jc, Lctx):
    NC, NS, L = 2, 16, 16
    NW = NC * NS
    rows_per = ROWS // NW
    RB = 32
    nbatch = rows_per // RB
    mesh = plsc.VectorSubcoreMesh(core_axis_name="c", subcore_axis_name="s")

    @functools.partial(
        pl.kernel, mesh=mesh,
        out_type=jax.ShapeDtypeStruct((ROWS, Lctx), jnp.float32),
        compiler_params=pltpu.CompilerParams(needs_layout_passes=False),
        scratch_types=[
            pltpu.VMEM((rows_per, Jc), jnp.float32),
            pltpu.VMEM((RB, Lctx), jnp.float32),
            pltpu.VMEM((RB, L), jnp.float32),
        ],
    )
    def k(logits_hbm, bias_hbm, lbuf, obuf, fscr):
        wid = lax.axis_index("s") * NC + lax.axis_index("c")
        base = wid * rows_per
        pltpu.sync_copy(logits_hbm.at[pl.ds(base, rows_per)], lbuf)

        lane = lax.broadcasted_iota(jnp.int32, (L,), 0)
        negv = jnp.full((L,), _NEG_INF, jnp.float32)
        zerov = jnp.zeros((L,), jnp.float32)
        i15 = jnp.full((L,), L - 1, jnp.int32)

        def batch_body(bi, carry):
            @plsc.parallel_loop(0, RB, step=1, unroll=2)
            def row_body(rr):
                frow = fscr.at[rr]

                def splat_f(vec, idxv):
                    frow[...] = vec
                    return plsc.load_gather(frow, [idxv])

                def splat_last_f(vec):
                    return splat_f(vec, i15)

                r = bi * RB + rr
                lvec = lbuf[r]
                t = lax.rem(base + r, T)
                mxv = splat_last_f(plsc.cummax(lvec))
                e = jnp.exp(lvec - mxv)
                sv = splat_last_f(plsc.cumsum(e))
                w = e / sv
                pcb = -8.0 * (1.0 - w)
                tail_ok = jnp.logical_and(lane >= _SINK, lane - _SINK <= t)
                masked = jnp.where(tail_ok, lvec, negv)
                skeys, _svals = plsc.sort_key_val(masked, lane,
                                                  descending=True)
                thr = splat_f(skeys, jnp.full((L,), _TOPK_MAIN - 1,
                                              jnp.int32))
                sel = jnp.logical_or(lane < _SINK, masked >= thr)
                vals = jnp.where(sel, zerov, pcb)
                cvals = jnp.where(lane * _CHUNK <= t, vals, negv)
                frow[...] = cvals
                for c in range(Jc):
                    idx = jnp.full((L,), c, jnp.int32)
                    splat = plsc.load_gather(frow, [idx])
                    for g in range(_CHUNK // L):
                        obuf[rr, pl.ds(c * _CHUNK + g * L, L)] = splat
                cb = lax.shift_right_logical(t, 7)
                splat_b = plsc.load_gather(frow, [jnp.full((L,), cb, jnp.int32)])
                cb128 = cb * _CHUNK
                for g in range(_CHUNK // L):
                    colv = lane + (cb128 + g * L)
                    obuf[rr, pl.ds(cb128 + g * L, L)] = jnp.where(
                        colv <= t, splat_b, negv)

            pltpu.sync_copy(obuf, bias_hbm.at[pl.ds(base + bi * RB, RB)])
            return carry

        lax.fori_loop(0, nbatch, batch_body, 0, unroll=False)

    return k


def kernel(Q, K, chunk_len, L_ctx, sink_blocks, Wq, Wk, ln_g, ln_b,
           Wu, bu, Wv, bv, Wo, bo):
    B, T, _ = Q.shape
    Jc = K.shape[1]
    Lctx = Jc * _CHUNK
    logits = _tc_logits(Q, K, Wq, Wk, ln_g, ln_b, Wu, bu, Wv, bv, Wo, bo)
    sc = _sc_bias_call(B * T, T, Jc, Lctx)
    bias = sc(logits.reshape(B * T, Jc)).reshape(B, T, Lctx)
    return logits, bias


# speedup vs baseline: 1.0859x; 1.0122x over previous
import functools
import jax
import jax.numpy as jnp
from jax import lax
from jax.experimental import pallas as pl
from jax.experimental.pallas import tpu as pltpu
from jax.experimental.pallas import tpu_sc as plsc

_SINK = 5
_TOPK_MAIN = 3
_CHUNK = 128
_TILE = 512
_NEG_INF = float("-inf")


def _tc_body(q_ref, k_ref, wq_ref, wk_ref, g_ref, bln_ref, wu_ref, wv_ref,
             bu_ref, bv_ref, wo_ref, bo_ref, logits_ref):
    H = wq_ref.shape[1]
    Jc = k_ref.shape[1]
    tile = q_ref.shape[1]
    f32 = jnp.float32

    Qt = q_ref[0]
    Kb = k_ref[0]
    qp = jnp.dot(Qt, wq_ref[...], preferred_element_type=f32)
    kp = jnp.dot(Kb, wk_ref[...], preferred_element_type=f32)

    g = g_ref[...]
    bln = bln_ref[...]
    Wu = wu_ref[...]
    Wv = wv_ref[...]
    g_q = g[:, :H]
    g_k = g[:, H:2 * H]
    g_m = g[:, 2 * H:]

    su = jnp.dot(g, Wu, preferred_element_type=f32)
    sv = jnp.dot(g, Wv, preferred_element_type=f32)
    cu = jnp.dot(bln, Wu, preferred_element_type=f32) + bu_ref[...]
    cv = jnp.dot(bln, Wv, preferred_element_type=f32) + bv_ref[...]

    ones_h = jnp.ones((1, H), dtype=f32)
    qp2 = qp * qp
    kp2 = kp * kp
    s1q = jnp.sum(qp, axis=1, keepdims=True)
    s2q = jnp.sum(qp2, axis=1, keepdims=True)
    dn = (((1,), (1,)), ((), ()))
    s1k = lax.dot_general(ones_h, kp, dn, preferred_element_type=f32)
    s2k = lax.dot_general(ones_h, kp2, dn, preferred_element_type=f32)
    G1 = lax.dot_general(qp, kp, dn, preferred_element_type=f32)
    G2 = lax.dot_general(qp2, kp2, dn, preferred_element_type=f32)
    inv_d = 1.0 / (3.0 * H)
    mu = (s1q + s1k + G1) * inv_d
    ex2 = (s2q + s2k + G2) * inv_d
    rstd = lax.rsqrt(ex2 - mu * mu + 1e-5)

    qs = qp * g_q
    ks = kp * g_k
    kg = kp * g_m
    Wu_q, Wu_k, Wu_m = Wu[:H], Wu[H:2 * H], Wu[2 * H:]
    Wv_q, Wv_k, Wv_m = Wv[:H], Wv[H:2 * H], Wv[2 * H:]
    Aq_u = jnp.dot(qs, Wu_q, preferred_element_type=f32)
    Aq_v = jnp.dot(qs, Wv_q, preferred_element_type=f32)
    Ak_u = jnp.dot(ks, Wu_k, preferred_element_type=f32)
    Ak_v = jnp.dot(ks, Wv_k, preferred_element_type=f32)

    wo = wo_ref[...]
    bos = bo_ref[0, 0]
    cols = []
    for j in range(Jc):
        xm = qp * kg[j:j + 1, :]
        Pu = jnp.dot(xm, Wu_m, preferred_element_type=f32)
        Pv = jnp.dot(xm, Wv_m, preferred_element_type=f32)
        rj = rstd[:, j:j + 1]
        muj = mu[:, j:j + 1]
        U = rj * (Aq_u + Ak_u[j:j + 1, :] + Pu - muj * su) + cu
        V = rj * (Aq_v + Ak_v[j:j + 1, :] + Pv - muj * sv) + cv
        geluV = 0.5 * V * (1.0 + lax.erf(V * 0.7071067811865476))
        Z = U * geluV
        cols.append(lax.dot_general(Z, wo, dn, preferred_element_type=f32))
    logits_ref[0] = jnp.concatenate(cols, axis=1) + bos


def _tc_logits(Q, K, Wq, Wk, ln_g, ln_b, Wu, bu, Wv, bv, Wo, bo):
    B, T, QD = Q.shape
    Jc = K.shape[1]
    KD = K.shape[2]
    H = Wq.shape[1]
    tile = _TILE
    grid = (B, T // tile)

    g2 = ln_g.reshape(1, -1)
    b2 = ln_b.reshape(1, -1)
    bu2 = bu.reshape(1, -1)
    bv2 = bv.reshape(1, -1)
    wo2 = Wo.reshape(1, -1)
    bo2 = bo.reshape(1, 1)

    full = lambda shape: pl.BlockSpec(shape, lambda b, t: (0,) * len(shape))
    return pl.pallas_call(
        _tc_body,
        grid=grid,
        in_specs=[
            pl.BlockSpec((1, tile, QD), lambda b, t: (b, t, 0)),
            pl.BlockSpec((1, Jc, KD), lambda b, t: (b, 0, 0)),
            full((QD, H)),
            full((KD, H)),
            full((1, 3 * H)),
            full((1, 3 * H)),
            full((3 * H, H)),
            full((3 * H, H)),
            full((1, H)),
            full((1, H)),
            full((1, H)),
            full((1, 1)),
        ],
        out_specs=pl.BlockSpec((1, tile, Jc), lambda b, t: (b, t, 0)),
        out_shape=jax.ShapeDtypeStruct((B, T, Jc), jnp.float32),
    )(Q, K, Wq, Wk, g2, b2, Wu, Wv, bu2, bv2, wo2, bo2)


def _sc_bias_call(ROWS, T, Jc, Lctx):
    NC, NS, L = 2, 16, 16
    NW = NC * NS
    rows_per = ROWS // NW
    RB = 32
    nbatch = rows_per // RB
    mesh = plsc.VectorSubcoreMesh(core_axis_name="c", subcore_axis_name="s")

    @functools.partial(
        pl.kernel, mesh=mesh,
        out_type=jax.ShapeDtypeStruct((ROWS, Lctx), jnp.float32),
        compiler_params=pltpu.CompilerParams(needs_layout_passes=False),
        scratch_types=[
            pltpu.VMEM((rows_per, Jc), jnp.float32),
            pltpu.VMEM((RB, Lctx), jnp.float32),
            pltpu.VMEM((RB, L), jnp.float32),
        ],
    )
    def k(logits_hbm, bias_hbm, lbuf, obuf, fscr):
        wid = lax.axis_index("s") * NC + lax.axis_index("c")
        base = wid * rows_per
        pltpu.sync_copy(logits_hbm.at[pl.ds(base, rows_per)], lbuf)

        lane = lax.broadcasted_iota(jnp.int32, (L,), 0)
        negv = jnp.full((L,), _NEG_INF, jnp.float32)
        zerov = jnp.zeros((L,), jnp.float32)
        i15 = jnp.full((L,), L - 1, jnp.int32)

        def batch_body(bi, carry):
            @plsc.parallel_loop(0, RB, step=1, unroll=2)
            def row_body(rr):
                frow = fscr.at[rr]

                def splat_f(vec, idxv):
                    frow[...] = vec
                    return plsc.load_gather(frow, [idxv])

                def splat_last_f(vec):
                    return splat_f(vec, i15)

                r = bi * RB + rr
                lvec = lbuf[r]
                t = lax.rem(base + r, T)
                mxv = splat_last_f(plsc.cummax(lvec))
                e = jnp.exp(lvec - mxv)
                sv = splat_last_f(plsc.cumsum(e))
                w = e / sv
                pcb = -8.0 * (1.0 - w)
                tail_ok = jnp.logical_and(lane >= _SINK, lane - _SINK <= t)
                masked = jnp.where(tail_ok, lvec, negv)
                skeys, _svals = plsc.sort_key_val(masked, lane,
                                                  descending=True)
                thr = splat_f(skeys, jnp.full((L,), _TOPK_MAIN - 1,
                                              jnp.int32))
                sel = jnp.logical_or(lane < _SINK, masked >= thr)
                vals = jnp.where(sel, zerov, pcb)
                cvals = jnp.where(lane * _CHUNK <= t, vals, negv)
                frow[...] = cvals
                for c in range(Jc):
                    idx = jnp.full((L,), c, jnp.int32)
                    splat = plsc.load_gather(frow, [idx])
                    for g in range(_CHUNK // L):
                        obuf[rr, pl.ds(c * _CHUNK + g * L, L)] = splat
                cb = lax.shift_right_logical(t, 7)
                splat_b = plsc.load_gather(frow, [jnp.full((L,), cb, jnp.int32)])
                cb128 = cb * _CHUNK
                for g in range(_CHUNK // L):
                    colv = lane + (cb128 + g * L)
                    obuf[rr, pl.ds(cb128 + g * L, L)] = jnp.where(
                        colv <= t, splat_b, negv)

            pltpu.sync_copy(obuf, bias_hbm.at[pl.ds(base + bi * RB, RB)])
            return carry

        lax.fori_loop(0, nbatch, batch_body, 0, unroll=False)

    return k


def kernel(Q, K, chunk_len, L_ctx, sink_blocks, Wq, Wk, ln_g, ln_b,
           Wu, bu, Wv, bv, Wo, bo):
    B, T, _ = Q.shape
    Jc = K.shape[1]
    Lctx = Jc * _CHUNK
    logits = _tc_logits(Q, K, Wq, Wk, ln_g, ln_b, Wu, bu, Wv, bv, Wo, bo)
    sc = _sc_bias_call(B * T, T, Jc, Lctx)
    bias = sc(logits.reshape(B * T, Jc)).reshape(B, T, Lctx)
    return logits, bias
